# baseline (speedup 1.0000x reference)
import functools

import jax
import jax.numpy as jnp
from jax import lax
from jax.experimental import pallas as pl
from jax.experimental.pallas import tpu as pltpu
from jax.experimental.pallas import tpu_sc as plsc

N = 10000
E = 160000
D = 256
DE = 16
H = 256
FFN = 512

HH = 128
N_PAD = 10240
ROWS_PER_TILE = N_PAD // 16
CH = 80
CHUNKS_PER_TILE = 125
EDGES_PER_TILE = CH * CHUNKS_PER_TILE


def _mm_split_body(x_ref, w_ref, b_ref, o_ref):
    acc = jnp.dot(x_ref[...], w_ref[0], preferred_element_type=jnp.float32)
    o_ref[0] = acc + b_ref[0]


def _mm_split(x, w3, b3, bm):
    m, k = x.shape
    out = pl.pallas_call(
        _mm_split_body,
        grid=(2, m // bm),
        in_specs=[
            pl.BlockSpec((bm, k), lambda c, i: (i, 0)),
            pl.BlockSpec((1, k, HH), lambda c, i: (c, 0, 0)),
            pl.BlockSpec((1, 1, HH), lambda c, i: (c, 0, 0)),
        ],
        out_specs=pl.BlockSpec((1, bm, HH), lambda c, i: (c, i, 0)),
        out_shape=jax.ShapeDtypeStruct((2, m, HH), jnp.float32),
    )(x, w3, b3)
    return out.reshape(2 * m, HH)


def _mm_dual_body(a_ref, w_ref, b_ref, o_ref):
    acc = jnp.dot(a_ref[0], w_ref[0, 0], preferred_element_type=jnp.float32)
    acc += jnp.dot(a_ref[1], w_ref[0, 1], preferred_element_type=jnp.float32)
    o_ref[0] = acc + b_ref[0]


def _mm_dual(a_flat, w4, b3, bm):
    np_ = a_flat.shape[0] // 2
    a3 = a_flat.reshape(2, np_, HH)
    out = pl.pallas_call(
        _mm_dual_body,
        grid=(2, np_ // bm),
        in_specs=[
            pl.BlockSpec((2, bm, HH), lambda c, i: (0, i, 0)),
            pl.BlockSpec((1, 2, HH, HH), lambda c, i: (c, 0, 0, 0)),
            pl.BlockSpec((1, 1, HH), lambda c, i: (c, 0, 0)),
        ],
        out_specs=pl.BlockSpec((1, bm, HH), lambda c, i: (c, i, 0)),
        out_shape=jax.ShapeDtypeStruct((2, np_, HH), jnp.float32),
    )(a3, w4, b3)
    return out.reshape(2 * np_, HH)


def _head_body(x_ref, a_ref, wox_ref, woa_ref, bo_ref, whd_ref, bhd_ref, o_ref):
    nh = jnp.dot(x_ref[...], wox_ref[...], preferred_element_type=jnp.float32)
    nh += jnp.dot(a_ref[0], woa_ref[0], preferred_element_type=jnp.float32)
    nh += jnp.dot(a_ref[1], woa_ref[1], preferred_element_type=jnp.float32)
    nh = jnp.maximum(nh + bo_ref[...], 0.0)
    o_ref[...] = jnp.dot(nh, whd_ref[...], preferred_element_type=jnp.float32) + bhd_ref[...]


def _head(x_pad, a_flat, wox, woa3, bo2, whd, bhd2, bm):
    np_ = x_pad.shape[0]
    a3 = a_flat.reshape(2, np_, HH)
    return pl.pallas_call(
        _head_body,
        grid=(np_ // bm,),
        in_specs=[
            pl.BlockSpec((bm, D), lambda i: (i, 0)),
            pl.BlockSpec((2, bm, HH), lambda i: (0, i, 0)),
            pl.BlockSpec((D, H), lambda i: (0, 0)),
            pl.BlockSpec((2, HH, H), lambda i: (0, 0, 0)),
            pl.BlockSpec((1, H), lambda i: (0, 0)),
            pl.BlockSpec((H, FFN), lambda i: (0, 0)),
            pl.BlockSpec((1, FFN), lambda i: (0, 0)),
        ],
        out_specs=pl.BlockSpec((bm, FFN), lambda i: (i, 0)),
        out_shape=jax.ShapeDtypeStruct((np_, FFN), jnp.float32),
    )(x_pad, a3, wox, woa3, bo2, whd, bhd2)


def _sc_body(write_h, table, edge, idx5, *rest):
    if write_h:
        (a_out, hout, r0, r1, r2, ebuf, i0v, i1v, i2v, acc,
         gs0, gs1, gs2, ss0, ss1, ss2, es) = rest
    else:
        (a_out, r0, r1, r2, ebuf, i0v, i1v, i2v, acc,
         gs0, gs1, gs2, ss0, ss1, ss2, es) = rest
        hout = None

    rbufs = (r0, r1, r2)
    ibufs = (i0v, i1v, i2v)
    gsems = (gs0, gs1, gs2)
    ssems = (ss0, ss1, ss2)

    c = lax.axis_index("c")
    s = lax.axis_index("s")

    def _run_graph(g):
        off = c * (2 * N_PAD) + g * N_PAD
        ebase = c * (2 * E) + g * E + s * EDGES_PER_TILE

        def _zrow(r, carry):
            for k in range(HH // 16):
                ebuf[r, pl.ds(k * 16, 16)] = jnp.zeros((16,), jnp.float32)
            return carry

        lax.fori_loop(0, CH, _zrow, 0)
        for j in range(ROWS_PER_TILE // CH):
            pltpu.sync_copy(ebuf, acc.at[pl.ds(s * ROWS_PER_TILE + j * CH, CH)])
        plsc.subcore_barrier()

        def _fetch_idx(i, u):
            pltpu.sync_copy(idx5.at[g, s, i], ibufs[u])
            for k in range(CH // 16):
                sl = pl.ds(k * 16, 16)
                ibufs[u][0, sl] = ibufs[u][0, sl] + off

        def _g_start(u):
            pltpu.async_copy(table.at[ibufs[u].at[0]], rbufs[u], gsems[u])

        def _g_wait(u):
            pltpu.make_async_copy(table.at[i0v.at[0]], rbufs[u], gsems[u]).wait()

        def _s_start(u):
            pltpu.async_copy(rbufs[u], acc.at[ibufs[u].at[1]], ssems[u], add=True)

        def _s_wait(u):
            pltpu.make_async_copy(r0, acc.at[i0v.at[1]], ssems[u]).wait()

        def _e_start(j):
            pltpu.async_copy(edge.at[pl.ds(ebase + j * CH, CH)], ebuf, es)

        def _e_wait():
            pltpu.make_async_copy(edge.at[pl.ds(ebase, CH)], ebuf, es).wait()

        def _unit(j, u):
            rb = rbufs[u]
            _g_wait(u)
            _e_wait()

            @plsc.parallel_loop(0, CH, unroll=4)
            def _crow(r):
                for k in range(HH // 16):
                    sl = pl.ds(k * 16, 16)
                    rb[r, sl] = jnp.maximum(rb[r, sl] + ebuf[r, sl], 0.0)

            nxt = j + 1
            if isinstance(nxt, int):
                if nxt < CHUNKS_PER_TILE:
                    _e_start(nxt)
            else:
                @pl.when(nxt < CHUNKS_PER_TILE)
                def _():
                    _e_start(nxt)

            if hout is not None:
                pltpu.sync_copy(rb, hout.at[pl.ds(ebase + j * CH, CH)])
            _s_start(u)

            up = (u + 2) % 3
            k = j + 2

            if isinstance(k, int):
                if k >= 3:
                    _s_wait(up)
                if k < CHUNKS_PER_TILE:
                    _fetch_idx(k, up)
                    _g_start(up)
            else:
                @pl.when(k >= 3)
                def _():
                    _s_wait(up)

                @pl.when(k < CHUNKS_PER_TILE)
                def _():
                    _fetch_idx(k, up)
                    _g_start(up)

        _fetch_idx(0, 0)
        _g_start(0)
        _fetch_idx(1, 1)
        _g_start(1)
        _e_start(0)

        def _step(t, carry):
            for u in range(3):
                _unit(3 * t + u, u)
            return carry

        nsteps = CHUNKS_PER_TILE // 3
        lax.fori_loop(0, nsteps, _step, 0)
        _unit(3 * nsteps, 0)
        _unit(3 * nsteps + 1, 1)
        _s_wait(1)

        plsc.subcore_barrier()
        pltpu.sync_copy(
            acc.at[pl.ds(s * ROWS_PER_TILE, ROWS_PER_TILE)],
            a_out.at[pl.ds(off + s * ROWS_PER_TILE, ROWS_PER_TILE)],
        )

    _run_graph(0)
    _run_graph(1)


@functools.lru_cache(maxsize=None)
def _sc_hop(write_h):
    mesh = plsc.VectorSubcoreMesh(core_axis_name="c", subcore_axis_name="s")
    out_type = [jax.ShapeDtypeStruct((4 * N_PAD, HH), jnp.float32)]
    if write_h:
        out_type.append(jax.ShapeDtypeStruct((4 * E, HH), jnp.float32))
    return pl.kernel(
        functools.partial(_sc_body, write_h),
        out_type=out_type,
        mesh=mesh,
        scratch_types=[
            pltpu.VMEM((CH, HH), jnp.float32),
            pltpu.VMEM((CH, HH), jnp.float32),
            pltpu.VMEM((CH, HH), jnp.float32),
            pltpu.VMEM((CH, HH), jnp.float32),
            pltpu.VMEM((2, CH), jnp.int32),
            pltpu.VMEM((2, CH), jnp.int32),
            pltpu.VMEM((2, CH), jnp.int32),
            pltpu.VMEM_SHARED((N_PAD, HH), jnp.float32),
            pltpu.SemaphoreType.DMA,
            pltpu.SemaphoreType.DMA,
            pltpu.SemaphoreType.DMA,
            pltpu.SemaphoreType.DMA,
            pltpu.SemaphoreType.DMA,
            pltpu.SemaphoreType.DMA,
            pltpu.SemaphoreType.DMA,
        ],
    )


def kernel(x_i, edge_index_i, edge_attr_i, x_j, edge_index_j, edge_attr_j,
           W_i, b_i, W_h, b_h, W_o, b_o, W_head, b_head):
    wi_x3 = W_i[:D].reshape(D, 2, HH).transpose(1, 0, 2)
    wi_e3 = W_i[D:].reshape(DE, 2, HH).transpose(1, 0, 2)
    zb2 = jnp.zeros((2, 1, HH), jnp.float32)
    bi2 = b_i.reshape(2, 1, HH)
    wh4 = W_h.reshape(2, HH, 2, HH).transpose(2, 0, 1, 3)
    bh2 = b_h.reshape(2, 1, HH)
    wox = W_o[:D]
    woa3 = W_o[D:].reshape(2, HH, H)
    bo2 = b_o.reshape(1, H)
    bhd2 = b_head.reshape(1, FFN)

    def _idx(edge_index):
        src = edge_index[0].reshape(16, CHUNKS_PER_TILE, CH)
        dst = edge_index[1].reshape(16, CHUNKS_PER_TILE, CH)
        return jnp.stack([src, dst], axis=2)

    idx5 = jnp.stack([_idx(edge_index_i), _idx(edge_index_j)])
    x_cat = jnp.concatenate([
        jnp.pad(x_i, ((0, N_PAD - N), (0, 0))),
        jnp.pad(x_j, ((0, N_PAD - N), (0, 0))),
    ])
    attr_cat = jnp.concatenate([edge_attr_i, edge_attr_j])

    xa = _mm_split(x_cat, wi_x3, zb2, 512)
    ea = _mm_split(attr_cat, wi_e3, bi2, 400)

    a_flat, h0 = _sc_hop(True)(xa, ea, idx5)
    for _ in range(2):
        m_flat = _mm_dual(a_flat, wh4, bh2, 512)
        (a_flat,) = _sc_hop(False)(m_flat, h0, idx5)

    out = _head(x_cat, a_flat, wox, woa3, bo2, W_head, bhd2, 512)
    zis = out[:N]
    zjs = out[N_PAD:N_PAD + N]
    return (zis, zjs, jnp.zeros((), jnp.float32))


# speedup vs baseline: 1.1284x; 1.1284x over previous
import functools

import jax
import jax.numpy as jnp
from jax import lax
from jax.experimental import pallas as pl
from jax.experimental.pallas import tpu as pltpu
from jax.experimental.pallas import tpu_sc as plsc

N = 10000
E = 160000
D = 256
DE = 16
H = 256
FFN = 512

HH = 128
N_PAD = 10240
ROWS_PER_TILE = N_PAD // 16
CH = 80
CHUNKS_PER_TILE = 125
EDGES_PER_TILE = CH * CHUNKS_PER_TILE


def _mm_split_body(x_ref, w_ref, b_ref, o_ref):
    acc = jnp.dot(x_ref[...], w_ref[0], preferred_element_type=jnp.float32)
    o_ref[0] = acc + b_ref[0]


def _mm_split(x, w3, b3, bm):
    m, k = x.shape
    out = pl.pallas_call(
        _mm_split_body,
        grid=(2, m // bm),
        in_specs=[
            pl.BlockSpec((bm, k), lambda c, i: (i, 0)),
            pl.BlockSpec((1, k, HH), lambda c, i: (c, 0, 0)),
            pl.BlockSpec((1, 1, HH), lambda c, i: (c, 0, 0)),
        ],
        out_specs=pl.BlockSpec((1, bm, HH), lambda c, i: (c, i, 0)),
        out_shape=jax.ShapeDtypeStruct((2, m, HH), jnp.float32),
    )(x, w3, b3)
    return out.reshape(2 * m, HH)


def _mm_dual_body(a_ref, w_ref, b_ref, o_ref):
    acc = jnp.dot(a_ref[0], w_ref[0, 0], preferred_element_type=jnp.float32)
    acc += jnp.dot(a_ref[1], w_ref[0, 1], preferred_element_type=jnp.float32)
    o_ref[0] = acc + b_ref[0]


def _mm_dual(a_flat, w4, b3, bm):
    np_ = a_flat.shape[0] // 2
    a3 = a_flat.reshape(2, np_, HH)
    out = pl.pallas_call(
        _mm_dual_body,
        grid=(2, np_ // bm),
        in_specs=[
            pl.BlockSpec((2, bm, HH), lambda c, i: (0, i, 0)),
            pl.BlockSpec((1, 2, HH, HH), lambda c, i: (c, 0, 0, 0)),
            pl.BlockSpec((1, 1, HH), lambda c, i: (c, 0, 0)),
        ],
        out_specs=pl.BlockSpec((1, bm, HH), lambda c, i: (c, i, 0)),
        out_shape=jax.ShapeDtypeStruct((2, np_, HH), jnp.float32),
    )(a3, w4, b3)
    return out.reshape(2 * np_, HH)


def _head_body(x_ref, a_ref, wox_ref, woa_ref, bo_ref, whd_ref, bhd_ref, o_ref):
    nh = jnp.dot(x_ref[...], wox_ref[...], preferred_element_type=jnp.float32)
    nh += jnp.dot(a_ref[0], woa_ref[0], preferred_element_type=jnp.float32)
    nh += jnp.dot(a_ref[1], woa_ref[1], preferred_element_type=jnp.float32)
    nh = jnp.maximum(nh + bo_ref[...], 0.0)
    o_ref[...] = jnp.dot(nh, whd_ref[...], preferred_element_type=jnp.float32) + bhd_ref[...]


def _head(x_pad, a_flat, wox, woa3, bo2, whd, bhd2, bm):
    np_ = x_pad.shape[0]
    a3 = a_flat.reshape(2, np_, HH)
    return pl.pallas_call(
        _head_body,
        grid=(np_ // bm,),
        in_specs=[
            pl.BlockSpec((bm, D), lambda i: (i, 0)),
            pl.BlockSpec((2, bm, HH), lambda i: (0, i, 0)),
            pl.BlockSpec((D, H), lambda i: (0, 0)),
            pl.BlockSpec((2, HH, H), lambda i: (0, 0, 0)),
            pl.BlockSpec((1, H), lambda i: (0, 0)),
            pl.BlockSpec((H, FFN), lambda i: (0, 0)),
            pl.BlockSpec((1, FFN), lambda i: (0, 0)),
        ],
        out_specs=pl.BlockSpec((bm, FFN), lambda i: (i, 0)),
        out_shape=jax.ShapeDtypeStruct((np_, FFN), jnp.float32),
    )(x_pad, a3, wox, woa3, bo2, whd, bhd2)


def _sc_body(write_h, table, edge, idx4, *rest):
    if write_h:
        (a_out, hout, r0, r1, r2, ebuf, i0v, i1v, i2v, acc,
         gs0, gs1, gs2, ss0, ss1, ss2, es, is0, is1, is2) = rest
    else:
        (a_out, r0, r1, r2, ebuf, i0v, i1v, i2v, acc,
         gs0, gs1, gs2, ss0, ss1, ss2, es, is0, is1, is2) = rest
        hout = None

    rbufs = (r0, r1, r2)
    ibufs = (i0v, i1v, i2v)
    gsems = (gs0, gs1, gs2)
    ssems = (ss0, ss1, ss2)
    isems = (is0, is1, is2)

    c = lax.axis_index("c")
    s = lax.axis_index("s")
    off = c * N_PAD
    ebase = c * E + s * EDGES_PER_TILE

    def _zrow(r, carry):
        for k in range(HH // 16):
            ebuf[r, pl.ds(k * 16, 16)] = jnp.zeros((16,), jnp.float32)
        return carry

    lax.fori_loop(0, CH, _zrow, 0)
    for j in range(ROWS_PER_TILE // CH):
        pltpu.sync_copy(ebuf, acc.at[pl.ds(s * ROWS_PER_TILE + j * CH, CH)])
    plsc.subcore_barrier()

    def _fetch_idx_start(i, u):
        pltpu.async_copy(idx4.at[s, i], ibufs[u], isems[u])

    def _fetch_idx_wait(u):
        pltpu.make_async_copy(idx4.at[s, 0], ibufs[u], isems[u]).wait()

    def _adjust(u):
        for k in range(CH // 16):
            sl = pl.ds(k * 16, 16)
            ibufs[u][0, sl] = ibufs[u][0, sl] + off

    def _g_start(u):
        pltpu.async_copy(table.at[ibufs[u].at[0]], rbufs[u], gsems[u])

    def _g_wait(u):
        pltpu.make_async_copy(table.at[i0v.at[0]], rbufs[u], gsems[u]).wait()

    def _s_start(u):
        pltpu.async_copy(rbufs[u], acc.at[ibufs[u].at[1]], ssems[u], add=True)

    def _s_wait(u):
        pltpu.make_async_copy(r0, acc.at[i0v.at[1]], ssems[u]).wait()

    def _e_start(j):
        pltpu.async_copy(edge.at[pl.ds(ebase + j * CH, CH)], ebuf, es)

    def _e_wait():
        pltpu.make_async_copy(edge.at[pl.ds(ebase, CH)], ebuf, es).wait()

    def _unit(j, u):
        rb = rbufs[u]
        u1 = (u + 1) % 3
        nxt = j + 1

        if isinstance(nxt, int):
            if nxt < CHUNKS_PER_TILE:
                _fetch_idx_wait(u1)
                _adjust(u1)
                _g_start(u1)
        else:
            @pl.when(nxt < CHUNKS_PER_TILE)
            def _():
                _fetch_idx_wait(u1)
                _adjust(u1)
                _g_start(u1)

        _g_wait(u)
        _e_wait()

        @plsc.parallel_loop(0, CH, unroll=4)
        def _crow(r):
            for k in range(HH // 16):
                sl = pl.ds(k * 16, 16)
                rb[r, sl] = jnp.maximum(rb[r, sl] + ebuf[r, sl], 0.0)

        if isinstance(nxt, int):
            if nxt < CHUNKS_PER_TILE:
                _e_start(nxt)
        else:
            @pl.when(nxt < CHUNKS_PER_TILE)
            def _():
                _e_start(nxt)

        if hout is not None:
            pltpu.sync_copy(rb, hout.at[pl.ds(ebase + j * CH, CH)])
        _s_start(u)

        up = (u + 2) % 3
        k = j + 2

        if isinstance(k, int):
            if k >= 3:
                _s_wait(up)
            if k < CHUNKS_PER_TILE:
                _fetch_idx_start(k, up)
        else:
            @pl.when(k >= 3)
            def _():
                _s_wait(up)

            @pl.when(k < CHUNKS_PER_TILE)
            def _():
                _fetch_idx_start(k, up)

    _fetch_idx_start(0, 0)
    _fetch_idx_start(1, 1)
    _fetch_idx_wait(0)
    _adjust(0)
    _g_start(0)
    _e_start(0)

    def _step(t, carry):
        for u in range(3):
            _unit(3 * t + u, u)
        return carry

    nsteps = CHUNKS_PER_TILE // 3
    lax.fori_loop(0, nsteps, _step, 0)
    _unit(3 * nsteps, 0)
    _unit(3 * nsteps + 1, 1)
    _s_wait(1)

    plsc.subcore_barrier()
    pltpu.sync_copy(
        acc.at[pl.ds(s * ROWS_PER_TILE, ROWS_PER_TILE)],
        a_out.at[pl.ds(c * N_PAD + s * ROWS_PER_TILE, ROWS_PER_TILE)],
    )


@functools.lru_cache(maxsize=None)
def _sc_hop(write_h):
    mesh = plsc.VectorSubcoreMesh(core_axis_name="c", subcore_axis_name="s")
    out_type = [jax.ShapeDtypeStruct((2 * N_PAD, HH), jnp.float32)]
    if write_h:
        out_type.append(jax.ShapeDtypeStruct((2 * E, HH), jnp.float32))
    return pl.kernel(
        functools.partial(_sc_body, write_h),
        out_type=out_type,
        mesh=mesh,
        scratch_types=[
            pltpu.VMEM((CH, HH), jnp.float32),
            pltpu.VMEM((CH, HH), jnp.float32),
            pltpu.VMEM((CH, HH), jnp.float32),
            pltpu.VMEM((CH, HH), jnp.float32),
            pltpu.VMEM((2, CH), jnp.int32),
            pltpu.VMEM((2, CH), jnp.int32),
            pltpu.VMEM((2, CH), jnp.int32),
            pltpu.VMEM_SHARED((N_PAD, HH), jnp.float32),
            pltpu.SemaphoreType.DMA,
            pltpu.SemaphoreType.DMA,
            pltpu.SemaphoreType.DMA,
            pltpu.SemaphoreType.DMA,
            pltpu.SemaphoreType.DMA,
            pltpu.SemaphoreType.DMA,
            pltpu.SemaphoreType.DMA,
            pltpu.SemaphoreType.DMA,
            pltpu.SemaphoreType.DMA,
            pltpu.SemaphoreType.DMA,
        ],
    )


def kernel(x_i, edge_index_i, edge_attr_i, x_j, edge_index_j, edge_attr_j,
           W_i, b_i, W_h, b_h, W_o, b_o, W_head, b_head):
    wi_x3 = W_i[:D].reshape(D, 2, HH).transpose(1, 0, 2)
    wi_e3 = W_i[D:].reshape(DE, 2, HH).transpose(1, 0, 2)
    zb2 = jnp.zeros((2, 1, HH), jnp.float32)
    bi2 = b_i.reshape(2, 1, HH)
    wh4 = W_h.reshape(2, HH, 2, HH).transpose(2, 0, 1, 3)
    bh2 = b_h.reshape(2, 1, HH)
    wox = W_o[:D]
    woa3 = W_o[D:].reshape(2, HH, H)
    bo2 = b_o.reshape(1, H)
    bhd2 = b_head.reshape(1, FFN)

    def _prep(x, edge_index, edge_attr):
        src = edge_index[0].reshape(16, CHUNKS_PER_TILE, CH)
        dst = edge_index[1].reshape(16, CHUNKS_PER_TILE, CH)
        idx4 = jnp.stack([src, dst], axis=2)
        x_p = jnp.pad(x, ((0, N_PAD - N), (0, 0)))
        xa = _mm_split(x_p, wi_x3, zb2, 512)
        ea = _mm_split(edge_attr, wi_e3, bi2, 400)
        return x_p, idx4, xa, ea

    xi_p, idx4_i, xa_i, ea_i = _prep(x_i, edge_index_i, edge_attr_i)
    xj_p, idx4_j, xa_j, ea_j = _prep(x_j, edge_index_j, edge_attr_j)

    a_i, h0_i = _sc_hop(True)(xa_i, ea_i, idx4_i)
    a_j, h0_j = _sc_hop(True)(xa_j, ea_j, idx4_j)
    for _ in range(2):
        m_i = _mm_dual(a_i, wh4, bh2, 512)
        (a_i,) = _sc_hop(False)(m_i, h0_i, idx4_i)
        m_j = _mm_dual(a_j, wh4, bh2, 512)
        (a_j,) = _sc_hop(False)(m_j, h0_j, idx4_j)

    zis = _head(xi_p, a_i, wox, woa3, bo2, W_head, bhd2, 512)[:N]
    zjs = _head(xj_p, a_j, wox, woa3, bo2, W_head, bhd2, 512)[:N]
    return (zis, zjs, jnp.zeros((), jnp.float32))


# speedup vs baseline: 1.1288x; 1.0004x over previous
import functools

import jax
import jax.numpy as jnp
from jax import lax
from jax.experimental import pallas as pl
from jax.experimental.pallas import tpu as pltpu
from jax.experimental.pallas import tpu_sc as plsc

N = 10000
E = 160000
D = 256
DE = 16
H = 256
FFN = 512

HH = 128
N_PAD = 10240
ROWS_PER_TILE = N_PAD // 16
CH = 80
CHUNKS_PER_TILE = 125
EDGES_PER_TILE = CH * CHUNKS_PER_TILE


def _mm_split_body(x_ref, w_ref, b_ref, o_ref):
    acc = jnp.dot(x_ref[...], w_ref[0], preferred_element_type=jnp.float32)
    o_ref[0] = acc + b_ref[0]


def _mm_split(x, w3, b3, bm):
    m, k = x.shape
    out = pl.pallas_call(
        _mm_split_body,
        grid=(2, m // bm),
        in_specs=[
            pl.BlockSpec((bm, k), lambda c, i: (i, 0)),
            pl.BlockSpec((1, k, HH), lambda c, i: (c, 0, 0)),
            pl.BlockSpec((1, 1, HH), lambda c, i: (c, 0, 0)),
        ],
        out_specs=pl.BlockSpec((1, bm, HH), lambda c, i: (c, i, 0)),
        out_shape=jax.ShapeDtypeStruct((2, m, HH), jnp.float32),
    )(x, w3, b3)
    return out.reshape(2 * m, HH)


def _mm_dual_body(a_ref, w_ref, b_ref, o_ref):
    acc = jnp.dot(a_ref[0], w_ref[0, 0], preferred_element_type=jnp.float32)
    acc += jnp.dot(a_ref[1], w_ref[0, 1], preferred_element_type=jnp.float32)
    o_ref[0] = acc + b_ref[0]


def _mm_dual(a_flat, w4, b3, bm):
    np_ = a_flat.shape[0] // 2
    a3 = a_flat.reshape(2, np_, HH)
    out = pl.pallas_call(
        _mm_dual_body,
        grid=(2, np_ // bm),
        in_specs=[
            pl.BlockSpec((2, bm, HH), lambda c, i: (0, i, 0)),
            pl.BlockSpec((1, 2, HH, HH), lambda c, i: (c, 0, 0, 0)),
            pl.BlockSpec((1, 1, HH), lambda c, i: (c, 0, 0)),
        ],
        out_specs=pl.BlockSpec((1, bm, HH), lambda c, i: (c, i, 0)),
        out_shape=jax.ShapeDtypeStruct((2, np_, HH), jnp.float32),
    )(a3, w4, b3)
    return out.reshape(2 * np_, HH)


def _head_body(x_ref, a_ref, wox_ref, woa_ref, bo_ref, whd_ref, bhd_ref, o_ref):
    nh = jnp.dot(x_ref[...], wox_ref[...], preferred_element_type=jnp.float32)
    nh += jnp.dot(a_ref[0], woa_ref[0], preferred_element_type=jnp.float32)
    nh += jnp.dot(a_ref[1], woa_ref[1], preferred_element_type=jnp.float32)
    nh = jnp.maximum(nh + bo_ref[...], 0.0)
    o_ref[...] = jnp.dot(nh, whd_ref[...], preferred_element_type=jnp.float32) + bhd_ref[...]


def _head(x_pad, a_flat, wox, woa3, bo2, whd, bhd2, bm):
    np_ = x_pad.shape[0]
    a3 = a_flat.reshape(2, np_, HH)
    return pl.pallas_call(
        _head_body,
        grid=(np_ // bm,),
        in_specs=[
            pl.BlockSpec((bm, D), lambda i: (i, 0)),
            pl.BlockSpec((2, bm, HH), lambda i: (0, i, 0)),
            pl.BlockSpec((D, H), lambda i: (0, 0)),
            pl.BlockSpec((2, HH, H), lambda i: (0, 0, 0)),
            pl.BlockSpec((1, H), lambda i: (0, 0)),
            pl.BlockSpec((H, FFN), lambda i: (0, 0)),
            pl.BlockSpec((1, FFN), lambda i: (0, 0)),
        ],
        out_specs=pl.BlockSpec((bm, FFN), lambda i: (i, 0)),
        out_shape=jax.ShapeDtypeStruct((np_, FFN), jnp.float32),
    )(x_pad, a3, wox, woa3, bo2, whd, bhd2)


def _sc_body(write_h, table, edge, idx4, *rest):
    if write_h:
        (a_out, hout, r0, r1, r2, ebuf, i0v, i1v, i2v, acc,
         gs0, gs1, gs2, ss0, ss1, ss2, es, is0, is1, is2,
         ws0, ws1, ws2) = rest
    else:
        (a_out, r0, r1, r2, ebuf, i0v, i1v, i2v, acc,
         gs0, gs1, gs2, ss0, ss1, ss2, es, is0, is1, is2,
         ws0, ws1, ws2) = rest
        hout = None

    rbufs = (r0, r1, r2)
    ibufs = (i0v, i1v, i2v)
    gsems = (gs0, gs1, gs2)
    ssems = (ss0, ss1, ss2)
    isems = (is0, is1, is2)
    wsems = (ws0, ws1, ws2)

    c = lax.axis_index("c")
    s = lax.axis_index("s")
    off = c * N_PAD
    ebase = c * E + s * EDGES_PER_TILE

    def _zrow(r, carry):
        for k in range(HH // 16):
            ebuf[r, pl.ds(k * 16, 16)] = jnp.zeros((16,), jnp.float32)
        return carry

    lax.fori_loop(0, CH, _zrow, 0)
    for j in range(ROWS_PER_TILE // CH):
        pltpu.sync_copy(ebuf, acc.at[pl.ds(s * ROWS_PER_TILE + j * CH, CH)])
    plsc.subcore_barrier()

    def _fetch_idx_start(i, u):
        pltpu.async_copy(idx4.at[s, i], ibufs[u], isems[u])

    def _fetch_idx_wait(u):
        pltpu.make_async_copy(idx4.at[s, 0], ibufs[u], isems[u]).wait()

    def _adjust(u):
        for k in range(CH // 16):
            sl = pl.ds(k * 16, 16)
            ibufs[u][0, sl] = ibufs[u][0, sl] + off

    def _g_start(u):
        pltpu.async_copy(table.at[ibufs[u].at[0]], rbufs[u], gsems[u])

    def _g_wait(u):
        pltpu.make_async_copy(table.at[i0v.at[0]], rbufs[u], gsems[u]).wait()

    def _s_start(u):
        pltpu.async_copy(rbufs[u], acc.at[ibufs[u].at[1]], ssems[u], add=True)

    def _s_wait(u):
        pltpu.make_async_copy(r0, acc.at[i0v.at[1]], ssems[u]).wait()

    def _e_start(j):
        pltpu.async_copy(edge.at[pl.ds(ebase + j * CH, CH)], ebuf, es)

    def _e_wait():
        pltpu.make_async_copy(edge.at[pl.ds(ebase, CH)], ebuf, es).wait()

    def _h_start(j, u):
        pltpu.async_copy(rbufs[u], hout.at[pl.ds(ebase + j * CH, CH)], wsems[u])

    def _h_wait(u):
        pltpu.make_async_copy(r0, hout.at[pl.ds(ebase, CH)], wsems[u]).wait()

    def _unit(j, u):
        rb = rbufs[u]
        u1 = (u + 1) % 3
        nxt = j + 1

        if isinstance(nxt, int):
            if nxt < CHUNKS_PER_TILE:
                if hout is not None and j >= 2:
                    _h_wait(u1)
                _fetch_idx_wait(u1)
                _adjust(u1)
                _g_start(u1)
        else:
            if hout is not None:
                @pl.when((nxt < CHUNKS_PER_TILE) & (j >= 2))
                def _():
                    _h_wait(u1)

            @pl.when(nxt < CHUNKS_PER_TILE)
            def _():
                _fetch_idx_wait(u1)
                _adjust(u1)
                _g_start(u1)

        _g_wait(u)
        _e_wait()

        @plsc.parallel_loop(0, CH, unroll=8)
        def _crow(r):
            for k in range(HH // 16):
                sl = pl.ds(k * 16, 16)
                rb[r, sl] = jnp.maximum(rb[r, sl] + ebuf[r, sl], 0.0)

        if isinstance(nxt, int):
            if nxt < CHUNKS_PER_TILE:
                _e_start(nxt)
        else:
            @pl.when(nxt < CHUNKS_PER_TILE)
            def _():
                _e_start(nxt)

        if hout is not None:
            _h_start(j, u)
        _s_start(u)

        up = (u + 2) % 3
        k = j + 2

        if isinstance(k, int):
            if k >= 3:
                _s_wait(up)
            if k < CHUNKS_PER_TILE:
                _fetch_idx_start(k, up)
        else:
            @pl.when(k >= 3)
            def _():
                _s_wait(up)

            @pl.when(k < CHUNKS_PER_TILE)
            def _():
                _fetch_idx_start(k, up)

    _fetch_idx_start(0, 0)
    _fetch_idx_start(1, 1)
    _fetch_idx_wait(0)
    _adjust(0)
    _g_start(0)
    _e_start(0)

    def _step(t, carry):
        for u in range(3):
            _unit(3 * t + u, u)
        return carry

    nsteps = CHUNKS_PER_TILE // 3
    lax.fori_loop(0, nsteps, _step, 0)
    _unit(3 * nsteps, 0)
    _unit(3 * nsteps + 1, 1)
    _s_wait(1)
    if write_h:
        _h_wait(2)
        _h_wait(0)
        _h_wait(1)

    plsc.subcore_barrier()
    pltpu.sync_copy(
        acc.at[pl.ds(s * ROWS_PER_TILE, ROWS_PER_TILE)],
        a_out.at[pl.ds(c * N_PAD + s * ROWS_PER_TILE, ROWS_PER_TILE)],
    )


@functools.lru_cache(maxsize=None)
def _sc_hop(write_h):
    mesh = plsc.VectorSubcoreMesh(core_axis_name="c", subcore_axis_name="s")
    out_type = [jax.ShapeDtypeStruct((2 * N_PAD, HH), jnp.float32)]
    if write_h:
        out_type.append(jax.ShapeDtypeStruct((2 * E, HH), jnp.float32))
    return pl.kernel(
        functools.partial(_sc_body, write_h),
        out_type=out_type,
        mesh=mesh,
        scratch_types=[
            pltpu.VMEM((CH, HH), jnp.float32),
            pltpu.VMEM((CH, HH), jnp.float32),
            pltpu.VMEM((CH, HH), jnp.float32),
            pltpu.VMEM((CH, HH), jnp.float32),
            pltpu.VMEM((2, CH), jnp.int32),
            pltpu.VMEM((2, CH), jnp.int32),
            pltpu.VMEM((2, CH), jnp.int32),
            pltpu.VMEM_SHARED((N_PAD, HH), jnp.float32),
            pltpu.SemaphoreType.DMA,
            pltpu.SemaphoreType.DMA,
            pltpu.SemaphoreType.DMA,
            pltpu.SemaphoreType.DMA,
            pltpu.SemaphoreType.DMA,
            pltpu.SemaphoreType.DMA,
            pltpu.SemaphoreType.DMA,
            pltpu.SemaphoreType.DMA,
            pltpu.SemaphoreType.DMA,
            pltpu.SemaphoreType.DMA,
            pltpu.SemaphoreType.DMA,
            pltpu.SemaphoreType.DMA,
            pltpu.SemaphoreType.DMA,
        ],
    )


def kernel(x_i, edge_index_i, edge_attr_i, x_j, edge_index_j, edge_attr_j,
           W_i, b_i, W_h, b_h, W_o, b_o, W_head, b_head):
    wi_x3 = W_i[:D].reshape(D, 2, HH).transpose(1, 0, 2)
    wi_e3 = W_i[D:].reshape(DE, 2, HH).transpose(1, 0, 2)
    zb2 = jnp.zeros((2, 1, HH), jnp.float32)
    bi2 = b_i.reshape(2, 1, HH)
    wh4 = W_h.reshape(2, HH, 2, HH).transpose(2, 0, 1, 3)
    bh2 = b_h.reshape(2, 1, HH)
    wox = W_o[:D]
    woa3 = W_o[D:].reshape(2, HH, H)
    bo2 = b_o.reshape(1, H)
    bhd2 = b_head.reshape(1, FFN)

    def _prep(x, edge_index, edge_attr):
        src = edge_index[0].reshape(16, CHUNKS_PER_TILE, CH)
        dst = edge_index[1].reshape(16, CHUNKS_PER_TILE, CH)
        idx4 = jnp.stack([src, dst], axis=2)
        x_p = jnp.pad(x, ((0, N_PAD - N), (0, 0)))
        xa = _mm_split(x_p, wi_x3, zb2, 512)
        ea = _mm_split(edge_attr, wi_e3, bi2, 400)
        return x_p, idx4, xa, ea

    xi_p, idx4_i, xa_i, ea_i = _prep(x_i, edge_index_i, edge_attr_i)
    xj_p, idx4_j, xa_j, ea_j = _prep(x_j, edge_index_j, edge_attr_j)

    a_i, h0_i = _sc_hop(True)(xa_i, ea_i, idx4_i)
    a_j, h0_j = _sc_hop(True)(xa_j, ea_j, idx4_j)
    for _ in range(2):
        m_i = _mm_dual(a_i, wh4, bh2, 512)
        (a_i,) = _sc_hop(False)(m_i, h0_i, idx4_i)
        m_j = _mm_dual(a_j, wh4, bh2, 512)
        (a_j,) = _sc_hop(False)(m_j, h0_j, idx4_j)

    zis = _head(xi_p, a_i, wox, woa3, bo2, W_head, bhd2, 512)[:N]
    zjs = _head(xj_p, a_j, wox, woa3, bo2, W_head, bhd2, 512)[:N]
    return (zis, zjs, jnp.zeros((), jnp.float32))


# speedup vs baseline: 1.1371x; 1.0073x over previous
import functools

import jax
import jax.numpy as jnp
from jax import lax
from jax.experimental import pallas as pl
from jax.experimental.pallas import tpu as pltpu
from jax.experimental.pallas import tpu_sc as plsc

N = 10000
E = 160000
D = 256
DE = 16
H = 256
FFN = 512

HH = 128
N_PAD = 10240
ROWS_PER_TILE = N_PAD // 16
CH = 80
CHUNKS_PER_TILE = 125
EDGES_PER_TILE = CH * CHUNKS_PER_TILE


def _mm_split_body(x_ref, w_ref, b_ref, o_ref):
    acc = jnp.dot(x_ref[...], w_ref[0], preferred_element_type=jnp.float32)
    o_ref[0] = acc + b_ref[0]


def _mm_split(x, w3, b3, bm):
    m, k = x.shape
    out = pl.pallas_call(
        _mm_split_body,
        grid=(2, m // bm),
        in_specs=[
            pl.BlockSpec((bm, k), lambda c, i: (i, 0)),
            pl.BlockSpec((1, k, HH), lambda c, i: (c, 0, 0)),
            pl.BlockSpec((1, 1, HH), lambda c, i: (c, 0, 0)),
        ],
        out_specs=pl.BlockSpec((1, bm, HH), lambda c, i: (c, i, 0)),
        out_shape=jax.ShapeDtypeStruct((2, m, HH), jnp.float32),
    )(x, w3, b3)
    return out.reshape(2 * m, HH)


def _mm_dual_body(a_ref, w_ref, b_ref, o_ref):
    acc = jnp.dot(a_ref[0], w_ref[0, 0], preferred_element_type=jnp.float32)
    acc += jnp.dot(a_ref[1], w_ref[0, 1], preferred_element_type=jnp.float32)
    o_ref[0] = acc + b_ref[0]


def _mm_dual(a_flat, w4, b3, bm):
    np_ = a_flat.shape[0] // 2
    a3 = a_flat.reshape(2, np_, HH)
    out = pl.pallas_call(
        _mm_dual_body,
        grid=(2, np_ // bm),
        in_specs=[
            pl.BlockSpec((2, bm, HH), lambda c, i: (0, i, 0)),
            pl.BlockSpec((1, 2, HH, HH), lambda c, i: (c, 0, 0, 0)),
            pl.BlockSpec((1, 1, HH), lambda c, i: (c, 0, 0)),
        ],
        out_specs=pl.BlockSpec((1, bm, HH), lambda c, i: (c, i, 0)),
        out_shape=jax.ShapeDtypeStruct((2, np_, HH), jnp.float32),
    )(a3, w4, b3)
    return out.reshape(2 * np_, HH)


def _head_body(x_ref, a_ref, wox_ref, woa_ref, bo_ref, whd_ref, bhd_ref, o_ref):
    nh = jnp.dot(x_ref[...], wox_ref[...], preferred_element_type=jnp.float32)
    nh += jnp.dot(a_ref[0], woa_ref[0], preferred_element_type=jnp.float32)
    nh += jnp.dot(a_ref[1], woa_ref[1], preferred_element_type=jnp.float32)
    nh = jnp.maximum(nh + bo_ref[...], 0.0)
    o_ref[...] = jnp.dot(nh, whd_ref[...], preferred_element_type=jnp.float32) + bhd_ref[...]


def _head(x_pad, a_flat, wox, woa3, bo2, whd, bhd2, bm):
    np_ = x_pad.shape[0]
    a3 = a_flat.reshape(2, np_, HH)
    return pl.pallas_call(
        _head_body,
        grid=(N // bm,),
        in_specs=[
            pl.BlockSpec((bm, D), lambda i: (i, 0)),
            pl.BlockSpec((2, bm, HH), lambda i: (0, i, 0)),
            pl.BlockSpec((D, H), lambda i: (0, 0)),
            pl.BlockSpec((2, HH, H), lambda i: (0, 0, 0)),
            pl.BlockSpec((1, H), lambda i: (0, 0)),
            pl.BlockSpec((H, FFN), lambda i: (0, 0)),
            pl.BlockSpec((1, FFN), lambda i: (0, 0)),
        ],
        out_specs=pl.BlockSpec((bm, FFN), lambda i: (i, 0)),
        out_shape=jax.ShapeDtypeStruct((N, FFN), jnp.float32),
    )(x_pad, a3, wox, woa3, bo2, whd, bhd2)


def _sc_body(write_h, table, edge, idx4, *rest):
    if write_h:
        (a_out, hout, r0, r1, r2, ebuf, i0v, i1v, i2v, acc,
         gs0, gs1, gs2, ss0, ss1, ss2, es, is0, is1, is2,
         ws0, ws1, ws2) = rest
    else:
        (a_out, r0, r1, r2, ebuf, i0v, i1v, i2v, acc,
         gs0, gs1, gs2, ss0, ss1, ss2, es, is0, is1, is2,
         ws0, ws1, ws2) = rest
        hout = None

    rbufs = (r0, r1, r2)
    ibufs = (i0v, i1v, i2v)
    gsems = (gs0, gs1, gs2)
    ssems = (ss0, ss1, ss2)
    isems = (is0, is1, is2)
    wsems = (ws0, ws1, ws2)

    c = lax.axis_index("c")
    s = lax.axis_index("s")
    off = c * N_PAD
    ebase = c * E + s * EDGES_PER_TILE

    def _zrow(r, carry):
        for k in range(HH // 16):
            ebuf[r, pl.ds(k * 16, 16)] = jnp.zeros((16,), jnp.float32)
        return carry

    lax.fori_loop(0, CH, _zrow, 0)
    for j in range(ROWS_PER_TILE // CH):
        pltpu.sync_copy(ebuf, acc.at[pl.ds(s * ROWS_PER_TILE + j * CH, CH)])
    plsc.subcore_barrier()

    def _fetch_idx_start(i, u):
        pltpu.async_copy(idx4.at[s, i], ibufs[u], isems[u])

    def _fetch_idx_wait(u):
        pltpu.make_async_copy(idx4.at[s, 0], ibufs[u], isems[u]).wait()

    def _adjust(u):
        for k in range(CH // 16):
            sl = pl.ds(k * 16, 16)
            ibufs[u][0, sl] = ibufs[u][0, sl] + off

    def _g_start(u):
        pltpu.async_copy(table.at[ibufs[u].at[0]], rbufs[u], gsems[u])

    def _g_wait(u):
        pltpu.make_async_copy(table.at[i0v.at[0]], rbufs[u], gsems[u]).wait()

    def _s_start(u):
        pltpu.async_copy(rbufs[u], acc.at[ibufs[u].at[1]], ssems[u], add=True)

    def _s_wait(u):
        pltpu.make_async_copy(r0, acc.at[i0v.at[1]], ssems[u]).wait()

    def _e_start(j):
        pltpu.async_copy(edge.at[pl.ds(ebase + j * CH, CH)], ebuf, es)

    def _e_wait():
        pltpu.make_async_copy(edge.at[pl.ds(ebase, CH)], ebuf, es).wait()

    def _h_start(j, u):
        pltpu.async_copy(rbufs[u], hout.at[pl.ds(ebase + j * CH, CH)], wsems[u])

    def _h_wait(u):
        pltpu.make_async_copy(r0, hout.at[pl.ds(ebase, CH)], wsems[u]).wait()

    def _unit(j, u):
        rb = rbufs[u]
        u1 = (u + 1) % 3
        nxt = j + 1

        if isinstance(nxt, int):
            if nxt < CHUNKS_PER_TILE:
                if hout is not None and j >= 2:
                    _h_wait(u1)
                _fetch_idx_wait(u1)
                _adjust(u1)
                _g_start(u1)
        else:
            if hout is not None:
                @pl.when((nxt < CHUNKS_PER_TILE) & (j >= 2))
                def _():
                    _h_wait(u1)

            @pl.when(nxt < CHUNKS_PER_TILE)
            def _():
                _fetch_idx_wait(u1)
                _adjust(u1)
                _g_start(u1)

        _g_wait(u)
        _e_wait()

        @plsc.parallel_loop(0, CH, unroll=8)
        def _crow(r):
            for k in range(HH // 16):
                sl = pl.ds(k * 16, 16)
                rb[r, sl] = jnp.maximum(rb[r, sl] + ebuf[r, sl], 0.0)

        if isinstance(nxt, int):
            if nxt < CHUNKS_PER_TILE:
                _e_start(nxt)
        else:
            @pl.when(nxt < CHUNKS_PER_TILE)
            def _():
                _e_start(nxt)

        if hout is not None:
            _h_start(j, u)
        _s_start(u)

        up = (u + 2) % 3
        k = j + 2

        if isinstance(k, int):
            if k >= 3:
                _s_wait(up)
            if k < CHUNKS_PER_TILE:
                _fetch_idx_start(k, up)
        else:
            @pl.when(k >= 3)
            def _():
                _s_wait(up)

            @pl.when(k < CHUNKS_PER_TILE)
            def _():
                _fetch_idx_start(k, up)

    _fetch_idx_start(0, 0)
    _fetch_idx_start(1, 1)
    _fetch_idx_wait(0)
    _adjust(0)
    _g_start(0)
    _e_start(0)

    def _step(t, carry):
        for u in range(3):
            _unit(3 * t + u, u)
        return carry

    nsteps = CHUNKS_PER_TILE // 3
    lax.fori_loop(0, nsteps, _step, 0)
    _unit(3 * nsteps, 0)
    _unit(3 * nsteps + 1, 1)
    _s_wait(1)
    if write_h:
        _h_wait(2)
        _h_wait(0)
        _h_wait(1)

    plsc.subcore_barrier()
    pltpu.sync_copy(
        acc.at[pl.ds(s * ROWS_PER_TILE, ROWS_PER_TILE)],
        a_out.at[pl.ds(c * N_PAD + s * ROWS_PER_TILE, ROWS_PER_TILE)],
    )


@functools.lru_cache(maxsize=None)
def _sc_hop(write_h):
    mesh = plsc.VectorSubcoreMesh(core_axis_name="c", subcore_axis_name="s")
    out_type = [jax.ShapeDtypeStruct((2 * N_PAD, HH), jnp.float32)]
    if write_h:
        out_type.append(jax.ShapeDtypeStruct((2 * E, HH), jnp.float32))
    return pl.kernel(
        functools.partial(_sc_body, write_h),
        out_type=out_type,
        mesh=mesh,
        scratch_types=[
            pltpu.VMEM((CH, HH), jnp.float32),
            pltpu.VMEM((CH, HH), jnp.float32),
            pltpu.VMEM((CH, HH), jnp.float32),
            pltpu.VMEM((CH, HH), jnp.float32),
            pltpu.VMEM((2, CH), jnp.int32),
            pltpu.VMEM((2, CH), jnp.int32),
            pltpu.VMEM((2, CH), jnp.int32),
            pltpu.VMEM_SHARED((N_PAD, HH), jnp.float32),
            pltpu.SemaphoreType.DMA,
            pltpu.SemaphoreType.DMA,
            pltpu.SemaphoreType.DMA,
            pltpu.SemaphoreType.DMA,
            pltpu.SemaphoreType.DMA,
            pltpu.SemaphoreType.DMA,
            pltpu.SemaphoreType.DMA,
            pltpu.SemaphoreType.DMA,
            pltpu.SemaphoreType.DMA,
            pltpu.SemaphoreType.DMA,
            pltpu.SemaphoreType.DMA,
            pltpu.SemaphoreType.DMA,
            pltpu.SemaphoreType.DMA,
        ],
    )


def kernel(x_i, edge_index_i, edge_attr_i, x_j, edge_index_j, edge_attr_j,
           W_i, b_i, W_h, b_h, W_o, b_o, W_head, b_head):
    wi_x3 = W_i[:D].reshape(D, 2, HH).transpose(1, 0, 2)
    wi_e3 = W_i[D:].reshape(DE, 2, HH).transpose(1, 0, 2)
    zb2 = jnp.zeros((2, 1, HH), jnp.float32)
    bi2 = b_i.reshape(2, 1, HH)
    wh4 = W_h.reshape(2, HH, 2, HH).transpose(2, 0, 1, 3)
    bh2 = b_h.reshape(2, 1, HH)
    wox = W_o[:D]
    woa3 = W_o[D:].reshape(2, HH, H)
    bo2 = b_o.reshape(1, H)
    bhd2 = b_head.reshape(1, FFN)

    def _prep(x, edge_index, edge_attr):
        src = edge_index[0].reshape(16, CHUNKS_PER_TILE, CH)
        dst = edge_index[1].reshape(16, CHUNKS_PER_TILE, CH)
        idx4 = jnp.stack([src, dst], axis=2)
        x_p = jnp.pad(x, ((0, N_PAD - N), (0, 0)))
        xa = _mm_split(x_p, wi_x3, zb2, 512)
        ea = _mm_split(edge_attr, wi_e3, bi2, 400)
        return x_p, idx4, xa, ea

    xi_p, idx4_i, xa_i, ea_i = _prep(x_i, edge_index_i, edge_attr_i)
    xj_p, idx4_j, xa_j, ea_j = _prep(x_j, edge_index_j, edge_attr_j)

    a_i, h0_i = _sc_hop(True)(xa_i, ea_i, idx4_i)
    a_j, h0_j = _sc_hop(True)(xa_j, ea_j, idx4_j)
    for _ in range(2):
        m_i = _mm_dual(a_i, wh4, bh2, 512)
        (a_i,) = _sc_hop(False)(m_i, h0_i, idx4_i)
        m_j = _mm_dual(a_j, wh4, bh2, 512)
        (a_j,) = _sc_hop(False)(m_j, h0_j, idx4_j)

    zis = _head(xi_p, a_i, wox, woa3, bo2, W_head, bhd2, 400)
    zjs = _head(xj_p, a_j, wox, woa3, bo2, W_head, bhd2, 400)
    return (zis, zjs, jnp.zeros((), jnp.float32))


# speedup vs baseline: 1.1437x; 1.0058x over previous
import functools

import jax
import jax.numpy as jnp
from jax import lax
from jax.experimental import pallas as pl
from jax.experimental.pallas import tpu as pltpu
from jax.experimental.pallas import tpu_sc as plsc

N = 10000
E = 160000
D = 256
DE = 16
H = 256
FFN = 512

HH = 128
N_PAD = 10240
ROWS_PER_TILE = N_PAD // 16
CH = 80
CHUNKS_PER_TILE = 125
EDGES_PER_TILE = CH * CHUNKS_PER_TILE


def _mm_split_body(x_ref, w_ref, b_ref, o_ref):
    acc = jnp.dot(x_ref[...], w_ref[0], preferred_element_type=jnp.float32)
    o_ref[0] = acc + b_ref[0]


def _mm_split(x, w3, b3, bm, mpad=None):
    m, k = x.shape
    mpad = m if mpad is None else mpad
    out = pl.pallas_call(
        _mm_split_body,
        grid=(2, m // bm),
        in_specs=[
            pl.BlockSpec((bm, k), lambda c, i: (i, 0)),
            pl.BlockSpec((1, k, HH), lambda c, i: (c, 0, 0)),
            pl.BlockSpec((1, 1, HH), lambda c, i: (c, 0, 0)),
        ],
        out_specs=pl.BlockSpec((1, bm, HH), lambda c, i: (c, i, 0)),
        out_shape=jax.ShapeDtypeStruct((2, mpad, HH), jnp.float32),
    )(x, w3, b3)
    return out.reshape(2 * mpad, HH)


def _mm_dual_body(a_ref, w_ref, b_ref, o_ref):
    acc = jnp.dot(a_ref[0], w_ref[0, 0], preferred_element_type=jnp.float32)
    acc += jnp.dot(a_ref[1], w_ref[0, 1], preferred_element_type=jnp.float32)
    o_ref[0] = acc + b_ref[0]


def _mm_dual(a_flat, w4, b3, bm):
    np_ = a_flat.shape[0] // 2
    a3 = a_flat.reshape(2, np_, HH)
    out = pl.pallas_call(
        _mm_dual_body,
        grid=(2, np_ // bm),
        in_specs=[
            pl.BlockSpec((2, bm, HH), lambda c, i: (0, i, 0)),
            pl.BlockSpec((1, 2, HH, HH), lambda c, i: (c, 0, 0, 0)),
            pl.BlockSpec((1, 1, HH), lambda c, i: (c, 0, 0)),
        ],
        out_specs=pl.BlockSpec((1, bm, HH), lambda c, i: (c, i, 0)),
        out_shape=jax.ShapeDtypeStruct((2, np_, HH), jnp.float32),
    )(a3, w4, b3)
    return out.reshape(2 * np_, HH)


def _head_body(x_ref, a_ref, wox_ref, woa_ref, bo_ref, whd_ref, bhd_ref, o_ref):
    nh = jnp.dot(x_ref[...], wox_ref[...], preferred_element_type=jnp.float32)
    nh += jnp.dot(a_ref[0], woa_ref[0], preferred_element_type=jnp.float32)
    nh += jnp.dot(a_ref[1], woa_ref[1], preferred_element_type=jnp.float32)
    nh = jnp.maximum(nh + bo_ref[...], 0.0)
    o_ref[...] = jnp.dot(nh, whd_ref[...], preferred_element_type=jnp.float32) + bhd_ref[...]


def _head(x_pad, a_flat, wox, woa3, bo2, whd, bhd2, bm):
    a3 = a_flat.reshape(2, a_flat.shape[0] // 2, HH)
    return pl.pallas_call(
        _head_body,
        grid=(N // bm,),
        in_specs=[
            pl.BlockSpec((bm, D), lambda i: (i, 0)),
            pl.BlockSpec((2, bm, HH), lambda i: (0, i, 0)),
            pl.BlockSpec((D, H), lambda i: (0, 0)),
            pl.BlockSpec((2, HH, H), lambda i: (0, 0, 0)),
            pl.BlockSpec((1, H), lambda i: (0, 0)),
            pl.BlockSpec((H, FFN), lambda i: (0, 0)),
            pl.BlockSpec((1, FFN), lambda i: (0, 0)),
        ],
        out_specs=pl.BlockSpec((bm, FFN), lambda i: (i, 0)),
        out_shape=jax.ShapeDtypeStruct((N, FFN), jnp.float32),
    )(x_pad, a3, wox, woa3, bo2, whd, bhd2)


def _sc_body(write_h, table, edge, idx4, *rest):
    if write_h:
        (a_out, hout, r0, r1, r2, ebuf, i0v, i1v, i2v, acc,
         gs0, gs1, gs2, ss0, ss1, ss2, es, is0, is1, is2,
         ws0, ws1, ws2) = rest
    else:
        (a_out, r0, r1, r2, ebuf, i0v, i1v, i2v, acc,
         gs0, gs1, gs2, ss0, ss1, ss2, es, is0, is1, is2,
         ws0, ws1, ws2) = rest
        hout = None

    rbufs = (r0, r1, r2)
    ibufs = (i0v, i1v, i2v)
    gsems = (gs0, gs1, gs2)
    ssems = (ss0, ss1, ss2)
    isems = (is0, is1, is2)
    wsems = (ws0, ws1, ws2)

    c = lax.axis_index("c")
    s = lax.axis_index("s")
    off = c * N_PAD
    ebase = c * E + s * EDGES_PER_TILE

    def _zrow(r, carry):
        for k in range(HH // 16):
            ebuf[r, pl.ds(k * 16, 16)] = jnp.zeros((16,), jnp.float32)
        return carry

    lax.fori_loop(0, CH, _zrow, 0)
    for j in range(ROWS_PER_TILE // CH):
        pltpu.sync_copy(ebuf, acc.at[pl.ds(s * ROWS_PER_TILE + j * CH, CH)])
    plsc.subcore_barrier()

    def _fetch_idx_start(i, u):
        pltpu.async_copy(idx4.at[s, i], ibufs[u], isems[u])

    def _fetch_idx_wait(u):
        pltpu.make_async_copy(idx4.at[s, 0], ibufs[u], isems[u]).wait()

    def _adjust(u):
        for k in range(CH // 16):
            sl = pl.ds(k * 16, 16)
            ibufs[u][0, sl] = ibufs[u][0, sl] + off

    def _g_start(u):
        pltpu.async_copy(table.at[ibufs[u].at[0]], rbufs[u], gsems[u])

    def _g_wait(u):
        pltpu.make_async_copy(table.at[i0v.at[0]], rbufs[u], gsems[u]).wait()

    def _s_start(u):
        pltpu.async_copy(rbufs[u], acc.at[ibufs[u].at[1]], ssems[u], add=True)

    def _s_wait(u):
        pltpu.make_async_copy(r0, acc.at[i0v.at[1]], ssems[u]).wait()

    def _e_start(j):
        pltpu.async_copy(edge.at[pl.ds(ebase + j * CH, CH)], ebuf, es)

    def _e_wait():
        pltpu.make_async_copy(edge.at[pl.ds(ebase, CH)], ebuf, es).wait()

    def _h_start(j, u):
        pltpu.async_copy(rbufs[u], hout.at[pl.ds(ebase + j * CH, CH)], wsems[u])

    def _h_wait(u):
        pltpu.make_async_copy(r0, hout.at[pl.ds(ebase, CH)], wsems[u]).wait()

    def _unit(j, u):
        rb = rbufs[u]
        u1 = (u + 1) % 3
        nxt = j + 1

        if isinstance(nxt, int):
            if nxt < CHUNKS_PER_TILE:
                if hout is not None and j >= 2:
                    _h_wait(u1)
                _fetch_idx_wait(u1)
                _adjust(u1)
                _g_start(u1)
        else:
            if hout is not None:
                @pl.when((nxt < CHUNKS_PER_TILE) & (j >= 2))
                def _():
                    _h_wait(u1)

            @pl.when(nxt < CHUNKS_PER_TILE)
            def _():
                _fetch_idx_wait(u1)
                _adjust(u1)
                _g_start(u1)

        _g_wait(u)
        _e_wait()

        @plsc.parallel_loop(0, CH, unroll=8)
        def _crow(r):
            for k in range(HH // 16):
                sl = pl.ds(k * 16, 16)
                rb[r, sl] = jnp.maximum(rb[r, sl] + ebuf[r, sl], 0.0)

        if isinstance(nxt, int):
            if nxt < CHUNKS_PER_TILE:
                _e_start(nxt)
        else:
            @pl.when(nxt < CHUNKS_PER_TILE)
            def _():
                _e_start(nxt)

        if hout is not None:
            _h_start(j, u)
        _s_start(u)

        up = (u + 2) % 3
        k = j + 2

        if isinstance(k, int):
            if k >= 3:
                _s_wait(up)
            if k < CHUNKS_PER_TILE:
                _fetch_idx_start(k, up)
        else:
            @pl.when(k >= 3)
            def _():
                _s_wait(up)

            @pl.when(k < CHUNKS_PER_TILE)
            def _():
                _fetch_idx_start(k, up)

    _fetch_idx_start(0, 0)
    _fetch_idx_start(1, 1)
    _fetch_idx_wait(0)
    _adjust(0)
    _g_start(0)
    _e_start(0)

    def _step(t, carry):
        for u in range(3):
            _unit(3 * t + u, u)
        return carry

    nsteps = CHUNKS_PER_TILE // 3
    lax.fori_loop(0, nsteps, _step, 0)
    _unit(3 * nsteps, 0)
    _unit(3 * nsteps + 1, 1)
    _s_wait(1)
    if write_h:
        _h_wait(2)
        _h_wait(0)
        _h_wait(1)

    plsc.subcore_barrier()
    pltpu.sync_copy(
        acc.at[pl.ds(s * ROWS_PER_TILE, ROWS_PER_TILE)],
        a_out.at[pl.ds(c * N_PAD + s * ROWS_PER_TILE, ROWS_PER_TILE)],
    )


@functools.lru_cache(maxsize=None)
def _sc_hop(write_h):
    mesh = plsc.VectorSubcoreMesh(core_axis_name="c", subcore_axis_name="s")
    out_type = [jax.ShapeDtypeStruct((2 * N_PAD, HH), jnp.float32)]
    if write_h:
        out_type.append(jax.ShapeDtypeStruct((2 * E, HH), jnp.float32))
    return pl.kernel(
        functools.partial(_sc_body, write_h),
        out_type=out_type,
        mesh=mesh,
        scratch_types=[
            pltpu.VMEM((CH, HH), jnp.float32),
            pltpu.VMEM((CH, HH), jnp.float32),
            pltpu.VMEM((CH, HH), jnp.float32),
            pltpu.VMEM((CH, HH), jnp.float32),
            pltpu.VMEM((2, CH), jnp.int32),
            pltpu.VMEM((2, CH), jnp.int32),
            pltpu.VMEM((2, CH), jnp.int32),
            pltpu.VMEM_SHARED((N_PAD, HH), jnp.float32),
            pltpu.SemaphoreType.DMA,
            pltpu.SemaphoreType.DMA,
            pltpu.SemaphoreType.DMA,
            pltpu.SemaphoreType.DMA,
            pltpu.SemaphoreType.DMA,
            pltpu.SemaphoreType.DMA,
            pltpu.SemaphoreType.DMA,
            pltpu.SemaphoreType.DMA,
            pltpu.SemaphoreType.DMA,
            pltpu.SemaphoreType.DMA,
            pltpu.SemaphoreType.DMA,
            pltpu.SemaphoreType.DMA,
            pltpu.SemaphoreType.DMA,
        ],
    )


def kernel(x_i, edge_index_i, edge_attr_i, x_j, edge_index_j, edge_attr_j,
           W_i, b_i, W_h, b_h, W_o, b_o, W_head, b_head):
    wi_x3 = W_i[:D].reshape(D, 2, HH).transpose(1, 0, 2)
    wi_e3 = W_i[D:].reshape(DE, 2, HH).transpose(1, 0, 2)
    zb2 = jnp.zeros((2, 1, HH), jnp.float32)
    bi2 = b_i.reshape(2, 1, HH)
    wh4 = W_h.reshape(2, HH, 2, HH).transpose(2, 0, 1, 3)
    bh2 = b_h.reshape(2, 1, HH)
    wox = W_o[:D]
    woa3 = W_o[D:].reshape(2, HH, H)
    bo2 = b_o.reshape(1, H)
    bhd2 = b_head.reshape(1, FFN)

    def _prep(x, edge_index, edge_attr):
        src = edge_index[0].reshape(16, CHUNKS_PER_TILE, CH)
        dst = edge_index[1].reshape(16, CHUNKS_PER_TILE, CH)
        idx4 = jnp.stack([src, dst], axis=2)
        xa = _mm_split(x, wi_x3, zb2, 400, N_PAD)
        ea = _mm_split(edge_attr, wi_e3, bi2, 400)
        return x, idx4, xa, ea

    xi_p, idx4_i, xa_i, ea_i = _prep(x_i, edge_index_i, edge_attr_i)
    xj_p, idx4_j, xa_j, ea_j = _prep(x_j, edge_index_j, edge_attr_j)

    a_i, h0_i = _sc_hop(True)(xa_i, ea_i, idx4_i)
    a_j, h0_j = _sc_hop(True)(xa_j, ea_j, idx4_j)
    for _ in range(2):
        m_i = _mm_dual(a_i, wh4, bh2, 512)
        (a_i,) = _sc_hop(False)(m_i, h0_i, idx4_i)
        m_j = _mm_dual(a_j, wh4, bh2, 512)
        (a_j,) = _sc_hop(False)(m_j, h0_j, idx4_j)

    zis = _head(xi_p, a_i, wox, woa3, bo2, W_head, bhd2, 400)
    zjs = _head(xj_p, a_j, wox, woa3, bo2, W_head, bhd2, 400)
    return (zis, zjs, jnp.zeros((), jnp.float32))
